# Initial kernel scaffold; baseline (speedup 1.0000x reference)
import jax
import jax.numpy as jnp
from jax.experimental import pallas as pl


def kernel(input_ids, embed_table):
    raise NotImplementedError("write your pallas kernel here")


# speedup vs baseline: 4.4835x; 4.4835x over previous
import functools

import jax
import jax.numpy as jnp
from jax import lax
from jax.experimental import pallas as pl
from jax.experimental.pallas import tpu as pltpu
from jax.experimental.pallas import tpu_sc as plsc

ROWS = 4096
COLS = 200
DIM = 4
N = ROWS * COLS

_info = plsc.get_sparse_core_info()
NC = _info.num_cores
NS = _info.num_subcores
NW = NC * NS
B_PER_W = N // NW
GROUPS = B_PER_W // 16


def _make_lookup():
    mesh = plsc.VectorSubcoreMesh(core_axis_name="c", subcore_axis_name="s")

    @functools.partial(
        pl.kernel,
        mesh=mesh,
        compiler_params=pltpu.CompilerParams(needs_layout_passes=False),
        out_type=jax.ShapeDtypeStruct((N * DIM,), jnp.float32),
        scratch_types=[
            pltpu.VMEM((32,), jnp.float32),
            pltpu.VMEM((B_PER_W,), jnp.int32),
            pltpu.VMEM((B_PER_W * DIM,), jnp.float32),
        ],
    )
    def lookup(ids_hbm, table_hbm, out_hbm, table_v, idx_v, rows_v):
        wid = lax.axis_index("s") * NC + lax.axis_index("c")
        base = wid * B_PER_W
        pltpu.sync_copy(table_hbm, table_v)
        pltpu.sync_copy(ids_hbm.at[pl.ds(base, B_PER_W)], idx_v)

        lane = lax.iota(jnp.int32, 16)
        pats = [lane * DIM + d for d in range(DIM)]

        def body(g, carry):
            ids16 = idx_v[pl.ds(g * 16, 16)]
            ids4 = ids16 * DIM
            g64v = jnp.full((16,), g * 64, dtype=jnp.int32)
            for d in range(DIM):
                vals = plsc.load_gather(table_v, [ids4 + d])
                plsc.store_scatter(rows_v, [g64v + pats[d]], vals)
            return carry

        lax.fori_loop(0, GROUPS, body, 0)
        pltpu.sync_copy(rows_v, out_hbm.at[pl.ds(base * DIM, B_PER_W * DIM)])

    return lookup


_lookup = _make_lookup()


def kernel(input_ids, embed_table):
    ids = input_ids.reshape(-1).astype(jnp.int32)
    out = _lookup(ids, embed_table.reshape(-1))
    return out.reshape(ROWS, COLS, DIM)


# speedup vs baseline: 4.6955x; 1.0473x over previous
import functools

import jax
import jax.numpy as jnp
from jax import lax
from jax.experimental import pallas as pl
from jax.experimental.pallas import tpu as pltpu
from jax.experimental.pallas import tpu_sc as plsc

ROWS = 4096
COLS = 200
DIM = 4
N = ROWS * COLS

_info = plsc.get_sparse_core_info()
NC = _info.num_cores
NS = _info.num_subcores
NW = NC * NS
B_PER_W = N // NW
GROUPS = B_PER_W // 16


def _make_lookup():
    mesh = plsc.VectorSubcoreMesh(core_axis_name="c", subcore_axis_name="s")

    @functools.partial(
        pl.kernel,
        mesh=mesh,
        compiler_params=pltpu.CompilerParams(needs_layout_passes=False),
        out_type=jax.ShapeDtypeStruct((N * DIM,), jnp.float32),
        scratch_types=[
            pltpu.VMEM((32,), jnp.float32),
            pltpu.VMEM((B_PER_W,), jnp.int32),
            pltpu.VMEM((B_PER_W * DIM,), jnp.float32),
        ],
    )
    def lookup(ids_hbm, table_hbm, out_hbm, table_v, idx_v, rows_v):
        wid = lax.axis_index("s") * NC + lax.axis_index("c")
        base = wid * B_PER_W
        pltpu.sync_copy(table_hbm, table_v)
        pltpu.sync_copy(ids_hbm.at[pl.ds(base, B_PER_W)], idx_v)

        lane = lax.iota(jnp.int32, 16)
        pats = [lane * DIM + d for d in range(DIM)]

        @plsc.parallel_loop(0, GROUPS, unroll=8)
        def body(g):
            ids16 = idx_v[pl.ds(g * 16, 16)]
            out_slice = rows_v.at[pl.ds(g * 64, 64)]
            for d in range(DIM):
                vals = plsc.load_gather(table_v.at[pl.ds(8 * d, 8)], [ids16])
                plsc.store_scatter(out_slice, [pats[d]], vals)

        pltpu.sync_copy(rows_v, out_hbm.at[pl.ds(base * DIM, B_PER_W * DIM)])

    return lookup


_lookup = _make_lookup()


def kernel(input_ids, embed_table):
    ids = input_ids.reshape(-1).astype(jnp.int32)
    out = _lookup(ids, embed_table.T.reshape(-1))
    return out.reshape(ROWS, COLS, DIM)


# speedup vs baseline: 4.7237x; 1.0060x over previous
import functools

import jax
import jax.numpy as jnp
from jax import lax
from jax.experimental import pallas as pl
from jax.experimental.pallas import tpu as pltpu
from jax.experimental.pallas import tpu_sc as plsc

ROWS = 4096
COLS = 200
DIM = 4
NUM_EMB = 8

_info = plsc.get_sparse_core_info()
NC = _info.num_cores
NS = _info.num_subcores
NW = NC * NS
R_PER_W = ROWS // NW
R_CHUNK = 64
N_CHUNK = R_PER_W // R_CHUNK

_OFFS = [*range(0, COLS - 15, 16)]
if _OFFS[-1] != COLS - 16:
    _OFFS.append(COLS - 16)


def _make_lookup():
    mesh = plsc.VectorSubcoreMesh(core_axis_name="c", subcore_axis_name="s")

    @functools.partial(
        pl.kernel,
        mesh=mesh,
        compiler_params=pltpu.CompilerParams(needs_layout_passes=False),
        out_type=jax.ShapeDtypeStruct((ROWS * COLS * DIM,), jnp.float32),
        scratch_types=[
            pltpu.VMEM((DIM * NUM_EMB,), jnp.float32),
            pltpu.VMEM((R_CHUNK, COLS), jnp.int32),
            pltpu.VMEM((R_CHUNK * COLS * DIM,), jnp.float32),
        ],
    )
    def lookup(ids_hbm, table_hbm, out_hbm, table_v, idx_v, rows_v):
        wid = lax.axis_index("s") * NC + lax.axis_index("c")
        r0 = wid * R_PER_W
        pltpu.sync_copy(table_hbm, table_v)

        lane = lax.iota(jnp.int32, 16)
        pats = [lane * DIM + d for d in range(DIM)]

        for h in range(N_CHUNK):
            pltpu.sync_copy(ids_hbm.at[pl.ds(r0 + h * R_CHUNK, R_CHUNK)], idx_v)

            @plsc.parallel_loop(0, R_CHUNK, unroll=2)
            def body(i):
                for off in _OFFS:
                    ids16 = idx_v[i, pl.ds(off, 16)]
                    dst = rows_v.at[pl.ds(i * (COLS * DIM) + off * DIM, 64)]
                    for d in range(DIM):
                        vals = plsc.load_gather(
                            table_v.at[pl.ds(NUM_EMB * d, NUM_EMB)], [ids16]
                        )
                        plsc.store_scatter(dst, [pats[d]], vals)

            pltpu.sync_copy(
                rows_v,
                out_hbm.at[
                    pl.ds((r0 + h * R_CHUNK) * COLS * DIM, R_CHUNK * COLS * DIM)
                ],
            )

    return lookup


_lookup = _make_lookup()


def kernel(input_ids, embed_table):
    out = _lookup(input_ids.astype(jnp.int32), embed_table.T.reshape(-1))
    return out.reshape(ROWS, COLS, DIM)


# speedup vs baseline: 6.4675x; 1.3692x over previous
import functools

import jax
import jax.numpy as jnp
from jax import lax
from jax.experimental import pallas as pl
from jax.experimental.pallas import tpu as pltpu
from jax.experimental.pallas import tpu_sc as plsc

ROWS = 4096
COLS = 200
DIM = 4
NUM_EMB = 8

_info = plsc.get_sparse_core_info()
NC = _info.num_cores
NS = _info.num_subcores
NW = NC * NS
R_PER_W = ROWS // NW
R_CHUNK = 64
N_CHUNK = R_PER_W // R_CHUNK

_OFFS = [*range(0, COLS - 15, 16)]
if _OFFS[-1] != COLS - 16:
    _OFFS.append(COLS - 16)


def _make_lookup():
    mesh = plsc.VectorSubcoreMesh(core_axis_name="c", subcore_axis_name="s")

    @functools.partial(
        pl.kernel,
        mesh=mesh,
        compiler_params=pltpu.CompilerParams(
            needs_layout_passes=False,
            use_tc_tiling_on_sc=False,
        ),
        out_type=jax.ShapeDtypeStruct((ROWS, COLS, DIM), jnp.float32),
        scratch_types=[
            pltpu.VMEM((DIM * NUM_EMB,), jnp.float32),
            pltpu.VMEM((R_CHUNK, COLS), jnp.int32),
            pltpu.VMEM((R_CHUNK, COLS, DIM), jnp.float32),
        ],
    )
    def lookup(ids_hbm, table_hbm, out_hbm, table_v, idx_v, out_v):
        wid = lax.axis_index("s") * NC + lax.axis_index("c")
        r0 = wid * R_PER_W
        pltpu.sync_copy(table_hbm, table_v)

        lane = lax.iota(jnp.int32, 16)
        cols = [lane + off for off in _OFFS]
        dims = [jnp.full((16,), d, dtype=jnp.int32) for d in range(DIM)]

        for h in range(N_CHUNK):
            pltpu.sync_copy(ids_hbm.at[pl.ds(r0 + h * R_CHUNK, R_CHUNK)], idx_v)

            @plsc.parallel_loop(0, R_CHUNK, unroll=2)
            def body(i):
                dst = out_v.at[i]
                for c, off in enumerate(_OFFS):
                    ids16 = idx_v[i, pl.ds(off, 16)]
                    for d in range(DIM):
                        vals = plsc.load_gather(
                            table_v.at[pl.ds(NUM_EMB * d, NUM_EMB)], [ids16]
                        )
                        plsc.store_scatter(dst, [cols[c], dims[d]], vals)

            pltpu.sync_copy(out_v, out_hbm.at[pl.ds(r0 + h * R_CHUNK, R_CHUNK)])

    return lookup


_lookup = _make_lookup()


def kernel(input_ids, embed_table):
    return _lookup(input_ids.astype(jnp.int32), embed_table.T.reshape(-1))
